# baseline (speedup 1.0000x reference)
import dataclasses
import functools

import jax
import jax.numpy as jnp
from jax import lax
from jax.experimental import pallas as pl
from jax.experimental.pallas import tpu as pltpu
from jax.experimental.pallas import tpu_sc as plsc

N = 10000
D = 128
DH = 64
E = 320000
NEG_SLOPE = 0.2

NSUB = 16
NCORE = 2
B = 128
EDGES_PER_TILE = E // NSUB
NCHUNK = -(-EDGES_PER_TILE // B)
EPT_PAD = NCHUNK * B
NBUF = 4
NPAD = 10240
ROWS_PER_TILE = NPAD // NSUB
ROWS_LAST = N - 15 * ROWS_PER_TILE


def _dense_in_body(x_ref, W_ref, asrc_ref, adst_ref, h2_ref, as_ref, ad_ref):
    h = jnp.dot(x_ref[...], W_ref[...], preferred_element_type=jnp.float32)
    h2_ref[0] = h[:, :DH]
    h2_ref[1] = h[:, DH:]
    as_ref[...] = jnp.sum(h * asrc_ref[...][None, :], axis=1, keepdims=True)
    ad_ref[...] = jnp.sum(h * adst_ref[...][None, :], axis=1, keepdims=True)


def _dense_in_fused_body(o1_ref, o2_ref, W_ref, asrc_ref, adst_ref,
                         h2_ref, as_ref, ad_ref):
    xv = (o1_ref[...] + o2_ref[...]) * 0.5
    h = jnp.dot(xv, W_ref[...], preferred_element_type=jnp.float32)
    h2_ref[0] = h[:, :DH]
    h2_ref[1] = h[:, DH:]
    as_ref[...] = jnp.sum(h * asrc_ref[...][None, :], axis=1, keepdims=True)
    ad_ref[...] = jnp.sum(h * adst_ref[...][None, :], axis=1, keepdims=True)


_DENSE_IN_OUT = [
    jax.ShapeDtypeStruct((NCORE, N, DH), jnp.float32),
    jax.ShapeDtypeStruct((N, 1), jnp.float32),
    jax.ShapeDtypeStruct((N, 1), jnp.float32),
]


def _dense_in(x, W, asrc, adst):
    return pl.pallas_call(_dense_in_body, out_shape=_DENSE_IN_OUT)(
        x, W, asrc, adst)


def _dense_in_fused(o1, o2, W, asrc, adst):
    return pl.pallas_call(_dense_in_fused_body, out_shape=_DENSE_IN_OUT)(
        o1, o2, W, asrc, adst)


def _dense_out_body(num_ref, den_ref, h2_ref, as_ref, ad_ref, b_ref, o_ref,
                    *, use_elu):
    a = as_ref[...] + ad_ref[...]
    a = jnp.where(a >= 0.0, a, a * NEG_SLOPE)
    ws = jnp.exp(a)
    inv = 1.0 / (den_ref[...] + ws + 1e-16)
    b = b_ref[...]
    for c in range(NCORE):
        o = (num_ref[c] + ws * h2_ref[c]) * inv + b[c * DH:(c + 1) * DH][None, :]
        if use_elu:
            o = jnp.where(o > 0.0, o, jnp.exp(o) - 1.0)
        o_ref[:, c * DH:(c + 1) * DH] = o


def _dense_out(num, den, h2, a_s, a_d, bias, use_elu):
    body = functools.partial(_dense_out_body, use_elu=use_elu)
    return pl.pallas_call(
        body, out_shape=jax.ShapeDtypeStruct((N, D), jnp.float32),
    )(num, den, h2, a_s, a_d, bias)


def _sc_agg_body(h2f_hbm, as_hbm, ad_hbm, src_hbm, dst_hbm,
                 numf_hbm, den_hbm,
                 srcb0, srcb1, srcb2, srcb3,
                 idxg0, idxg1, idxg2, idxg3,
                 idxs0, idxs1, idxs2, idxs3,
                 asv0, asv1, asv2, asv3,
                 adv0, adv1, adv2, adv3,
                 wb0, wb1, wb2, wb3,
                 wsc0, wsc1, wsc2, wsc3,
                 rows_g0, rows_g1, rows_g2, rows_g3,
                 rows_s0, rows_s1, rows_s2, rows_s3,
                 as_sh, ad_sh, num_sp, den_sp,
                 isem0, isem1, isem2, isem3,
                 asem0, asem1, asem2, asem3,
                 gsem0, gsem1, gsem2, gsem3,
                 ssem0, ssem1, ssem2, ssem3,
                 dsem0, dsem1, dsem2, dsem3):
    cid = lax.axis_index("c")
    sid = lax.axis_index("s")

    srcb = (srcb0, srcb1, srcb2, srcb3)
    idxg = (idxg0, idxg1, idxg2, idxg3)
    idxs = (idxs0, idxs1, idxs2, idxs3)
    asv = (asv0, asv1, asv2, asv3)
    adv = (adv0, adv1, adv2, adv3)
    wb = (wb0, wb1, wb2, wb3)
    wsc = (wsc0, wsc1, wsc2, wsc3)
    rows_g = (rows_g0, rows_g1, rows_g2, rows_g3)
    rows_s = (rows_s0, rows_s1, rows_s2, rows_s3)
    isem = (isem0, isem1, isem2, isem3)
    asem = (asem0, asem1, asem2, asem3)
    gsem = (gsem0, gsem1, gsem2, gsem3)
    ssem = (ssem0, ssem1, ssem2, ssem3)
    dsem = (dsem0, dsem1, dsem2, dsem3)

    zero16 = jnp.zeros((16,), jnp.float32)
    lane = lax.iota(jnp.int32, 16)
    nvec = B // 16
    ebase = sid * EPT_PAD

    @pl.when(sid == 0)
    def _tables():
        pltpu.sync_copy(as_hbm, as_sh)
        pltpu.sync_copy(ad_hbm, ad_sh)

    @pl.loop(0, B)
    def _zrow(r):
        for f in range(4):
            rows_g0[r, pl.ds(16 * f, 16)] = zero16

    for j in range(nvec):
        wb0[pl.ds(16 * j, 16)] = zero16

    row0 = sid * ROWS_PER_TILE
    for k in range(ROWS_PER_TILE // B):
        pltpu.sync_copy(rows_g0, num_sp.at[pl.ds(row0 + k * B, B)])
        pltpu.sync_copy(wb0, den_sp.at[pl.ds(row0 + k * B, B)])
    plsc.subcore_barrier()

    coff = jnp.full((16,), cid * N, jnp.int32)


    def stage_idx(b, c):
        pltpu.async_copy(src_hbm.at[pl.ds(ebase + c * B, B)], srcb[b], isem[b])
        pltpu.async_copy(dst_hbm.at[pl.ds(ebase + c * B, B)], idxs[b], isem[b])

    def stage_gather(b, c):
        pltpu.make_async_copy(src_hbm.at[pl.ds(ebase + c * B, B)],
                              srcb[b], isem[b]).wait()
        pltpu.make_async_copy(dst_hbm.at[pl.ds(ebase + c * B, B)],
                              idxs[b], isem[b]).wait()
        pltpu.async_copy(as_sh.at[srcb[b]], asv[b], asem[b])
        pltpu.async_copy(ad_sh.at[idxs[b]], adv[b], asem[b])
        for j in range(nvec):
            sl = pl.ds(16 * j, 16)
            idxg[b][sl] = srcb[b][sl] + coff
        pltpu.async_copy(h2f_hbm.at[idxg[b]], rows_g[b], gsem[b])

    def stage_w(b, c):
        pltpu.make_async_copy(as_sh.at[srcb[b]], asv[b], asem[b]).wait()
        pltpu.make_async_copy(ad_sh.at[idxs[b]], adv[b], asem[b]).wait()
        for j in range(nvec):
            sl = pl.ds(16 * j, 16)
            e = asv[b][sl] + adv[b][sl]
            e = jnp.where(e >= 0.0, e, e * NEG_SLOPE)
            w = jnp.exp(e)
            valid = (lane + (c * B + 16 * j)) < EDGES_PER_TILE
            wb[b][sl] = jnp.where(valid, w, 0.0)

    def stage_scale(b, c):
        pltpu.make_async_copy(h2f_hbm.at[idxg[b]], rows_g[b], gsem[b]).wait()
        for j in range(nvec):
            wv = wb[b][pl.ds(16 * j, 16)]
            for r in range(16):
                wr = jnp.full((16,), wv[r])
                row = 16 * j + r
                for f in range(4):
                    sl = pl.ds(16 * f, 16)
                    rows_s[b][row, sl] = rows_g[b][row, sl] * wr
        pltpu.async_copy(rows_s[b], num_sp.at[idxs[b]], ssem[b], add=True)

        @pl.when(cid == 0)
        def _():
            for j in range(nvec):
                sl = pl.ds(16 * j, 16)
                wsc[b][sl] = wb[b][sl]
            pltpu.async_copy(wsc[b], den_sp.at[idxs[b]], dsem[b], add=True)

    def wait_scatter(b):
        pltpu.make_async_copy(rows_s[b], num_sp.at[idxs[b]], ssem[b]).wait()

        @pl.when(cid == 0)
        def _():
            pltpu.make_async_copy(wsc[b], den_sp.at[idxs[b]], dsem[b]).wait()

    for c in range(3):
        stage_idx(c % NBUF, c)
    for c in range(2):
        stage_gather(c % NBUF, c)
    stage_w(0, 0)

    @pl.loop(0, (NCHUNK + NBUF - 1) // NBUF)
    def _grp(k):
        for b in range(NBUF):
            c = NBUF * k + b
            b1 = (b + 1) % NBUF
            b2 = (b + 2) % NBUF
            b3 = (b + 3) % NBUF

            @pl.when(c < NCHUNK)
            def _(c=c, b=b):
                stage_scale(b, c)

            @pl.when(c + 1 < NCHUNK)
            def _(c=c, b1=b1):
                stage_w(b1, c + 1)

            @pl.when(c + 2 < NCHUNK)
            def _(c=c, b2=b2):
                stage_gather(b2, c + 2)

            @pl.when(c + 3 < NCHUNK)
            def _(c=c, b3=b3):
                @pl.when(c + 3 >= NBUF)
                def _():
                    wait_scatter(b3)

                stage_idx(b3, c + 3)

    for b in range(NBUF):
        wait_scatter(b)

    plsc.subcore_barrier()

    out0 = cid * N + row0

    @pl.when(sid < NSUB - 1)
    def _wfull():
        pltpu.sync_copy(num_sp.at[pl.ds(row0, ROWS_PER_TILE)],
                        numf_hbm.at[pl.ds(out0, ROWS_PER_TILE)])

    @pl.when(sid == NSUB - 1)
    def _wlast():
        pltpu.sync_copy(num_sp.at[pl.ds(row0, ROWS_LAST)],
                        numf_hbm.at[pl.ds(out0, ROWS_LAST)])

    @pl.when(cid == 0)
    def _wden():
        @pl.when(sid < NSUB - 1)
        def _dfull():
            pltpu.sync_copy(den_sp.at[pl.ds(row0, ROWS_PER_TILE)],
                            den_hbm.at[pl.ds(row0, ROWS_PER_TILE)])

        @pl.when(sid == NSUB - 1)
        def _dlast():
            pltpu.sync_copy(den_sp.at[pl.ds(row0, ROWS_LAST)],
                            den_hbm.at[pl.ds(row0, ROWS_LAST)])


def _sc_agg(h2, a_s, a_d, src, dst):
    mesh = plsc.VectorSubcoreMesh(core_axis_name="c", subcore_axis_name="s")
    h2f = h2.reshape(NCORE * N, DH)
    cp = pltpu.CompilerParams()
    if "needs_layout_passes" in pltpu.CompilerParams.__dataclass_fields__:
        cp = dataclasses.replace(cp, needs_layout_passes=False)
    if "use_tc_tiling_on_sc" in pltpu.CompilerParams.__dataclass_fields__:
        cp = dataclasses.replace(cp, use_tc_tiling_on_sc=False)
    kern = pl.kernel(
        _sc_agg_body,
        out_type=[
            jax.ShapeDtypeStruct((NCORE * N, DH), jnp.float32),
            jax.ShapeDtypeStruct((N,), jnp.float32),
        ],
        mesh=mesh,
        scratch_types=(
            [pltpu.VMEM((B,), jnp.int32)] * NBUF
            + [pltpu.VMEM((B,), jnp.int32)] * NBUF
            + [pltpu.VMEM((B,), jnp.int32)] * NBUF
            + [pltpu.VMEM((B,), jnp.float32)] * NBUF
            + [pltpu.VMEM((B,), jnp.float32)] * NBUF
            + [pltpu.VMEM((B,), jnp.float32)] * NBUF
            + [pltpu.VMEM((B,), jnp.float32)] * NBUF
            + [pltpu.VMEM((B, DH), jnp.float32)] * NBUF
            + [pltpu.VMEM((B, DH), jnp.float32)] * NBUF
            + [
                pltpu.VMEM_SHARED((N,), jnp.float32),
                pltpu.VMEM_SHARED((N,), jnp.float32),
                pltpu.VMEM_SHARED((NPAD, DH), jnp.float32),
                pltpu.VMEM_SHARED((NPAD,), jnp.float32),
            ]
            + [pltpu.SemaphoreType.DMA] * (5 * NBUF)
        ),
        compiler_params=cp,
    )
    return kern(h2f, a_s, a_d, src, dst)


def _pad_edges(a):
    a2 = a.reshape(NSUB, EDGES_PER_TILE)
    a2 = jnp.pad(a2, ((0, 0), (0, EPT_PAD - EDGES_PER_TILE)))
    return a2.reshape(NSUB * EPT_PAD)


def _gat_layer(h2, a_s, a_d, src, dst, bias, use_elu):
    num, den = _sc_agg(h2, a_s.reshape(N), a_d.reshape(N), src, dst)
    num = num.reshape(NCORE, N, DH)
    den = den.reshape(N, 1)
    return _dense_out(num, den, h2, a_s, a_d, bias, use_elu)


def kernel(x, spatial_edge_index, gene_sim_edge_index,
           W_sp, a_src_sp, a_dst_sp, b_sp,
           W_gs, a_src_gs, a_dst_gs, b_gs,
           W_f, a_src_f, a_dst_f, b_f):
    sp_src = _pad_edges(spatial_edge_index[0])
    sp_dst = _pad_edges(spatial_edge_index[1])
    gs_src = _pad_edges(gene_sim_edge_index[0])
    gs_dst = _pad_edges(gene_sim_edge_index[1])

    h2_sp, as_sp, ad_sp = _dense_in(x, W_sp, a_src_sp, a_dst_sp)
    h2_gs, as_gs, ad_gs = _dense_in(x, W_gs, a_src_gs, a_dst_gs)

    o1 = _gat_layer(h2_sp, as_sp, ad_sp, sp_src, sp_dst, b_sp, True)
    o2 = _gat_layer(h2_gs, as_gs, ad_gs, gs_src, gs_dst, b_gs, True)

    h2_f, as_f, ad_f = _dense_in_fused(o1, o2, W_f, a_src_f, a_dst_f)
    return _gat_layer(h2_f, as_f, ad_f, sp_src, sp_dst, b_f, False)


# speedup vs baseline: 1.2841x; 1.2841x over previous
import dataclasses
import functools

import jax
import jax.numpy as jnp
from jax import lax
from jax.experimental import pallas as pl
from jax.experimental.pallas import tpu as pltpu
from jax.experimental.pallas import tpu_sc as plsc

N = 10000
D = 128
DH = 64
E = 320000
NEG_SLOPE = 0.2

NSUB = 16
NCORE = 2
B = 64
EDGES_PER_TILE = E // NSUB
NCHUNK = -(-EDGES_PER_TILE // B)
EPT_PAD = NCHUNK * B
NBUF = 3
NPAD = 10240
ROWS_PER_TILE = NPAD // NSUB
ROWS_LAST = N - 15 * ROWS_PER_TILE


def _project(x, W_ref, asrc_ref, adst_ref, h2_ref, as_ref, ad_ref):
    h = jnp.dot(x, W_ref[...], preferred_element_type=jnp.float32)
    h2_ref[0] = h[:, :DH]
    h2_ref[1] = h[:, DH:]
    as_ref[...] = jnp.sum(h * asrc_ref[...][None, :], axis=1, keepdims=True)
    ad_ref[...] = jnp.sum(h * adst_ref[...][None, :], axis=1, keepdims=True)


def _epilogue(num_ref, den_ref, h2_ref, as_ref, ad_ref, b_ref, use_elu):
    a = as_ref[...] + ad_ref[...]
    a = jnp.where(a >= 0.0, a, a * NEG_SLOPE)
    ws = jnp.exp(a)
    inv = 1.0 / (den_ref[...] + ws + 1e-16)
    b = b_ref[...]
    halves = []
    for c in range(NCORE):
        o = (num_ref[c] + ws * h2_ref[c]) * inv + b[c * DH:(c + 1) * DH][None, :]
        if use_elu:
            o = jnp.where(o > 0.0, o, jnp.exp(o) - 1.0)
        halves.append(o)
    return jnp.concatenate(halves, axis=1)


_DENSE_IN_OUT = [
    jax.ShapeDtypeStruct((NCORE, N, DH), jnp.float32),
    jax.ShapeDtypeStruct((N, 1), jnp.float32),
    jax.ShapeDtypeStruct((N, 1), jnp.float32),
]


def _dense_in2_body(x_ref, Wsp_ref, assp_ref, adsp_ref, Wgs_ref, asgs_ref,
                    adgs_ref, h2sp_ref, assp_o, adsp_o, h2gs_ref, asgs_o,
                    adgs_o):
    xv = x_ref[...]
    _project(xv, Wsp_ref, assp_ref, adsp_ref, h2sp_ref, assp_o, adsp_o)
    _project(xv, Wgs_ref, asgs_ref, adgs_ref, h2gs_ref, asgs_o, adgs_o)


def _dense_in2(x, W_sp, a_src_sp, a_dst_sp, W_gs, a_src_gs, a_dst_gs):
    return pl.pallas_call(_dense_in2_body, out_shape=_DENSE_IN_OUT * 2)(
        x, W_sp, a_src_sp, a_dst_sp, W_gs, a_src_gs, a_dst_gs)


def _dense_mid_body(num1_ref, den1_ref, h2sp_ref, assp_ref, adsp_ref, bsp_ref,
                    num2_ref, den2_ref, h2gs_ref, asgs_ref, adgs_ref, bgs_ref,
                    Wf_ref, asf_ref, adf_ref,
                    h2f_ref, asf_o, adf_o):
    o1 = _epilogue(num1_ref, den1_ref, h2sp_ref, assp_ref, adsp_ref, bsp_ref,
                   True)
    o2 = _epilogue(num2_ref, den2_ref, h2gs_ref, asgs_ref, adgs_ref, bgs_ref,
                   True)
    xf = (o1 + o2) * 0.5
    _project(xf, Wf_ref, asf_ref, adf_ref, h2f_ref, asf_o, adf_o)


_BN = 2000


def _dense_mid(num1, den1, h2_sp, as_sp, ad_sp, b_sp,
               num2, den2, h2_gs, as_gs, ad_gs, b_gs,
               W_f, a_src_f, a_dst_f):
    nodes2 = pl.BlockSpec((NCORE, _BN, DH), lambda i: (0, i, 0))
    nodes1 = pl.BlockSpec((_BN, 1), lambda i: (i, 0))
    full_w = pl.BlockSpec((D, D), lambda i: (0, 0))
    full_v = pl.BlockSpec((D,), lambda i: (0,))
    return pl.pallas_call(
        _dense_mid_body,
        grid=(N // _BN,),
        in_specs=[nodes2, nodes1, nodes2, nodes1, nodes1, full_v,
                  nodes2, nodes1, nodes2, nodes1, nodes1, full_v,
                  full_w, full_v, full_v],
        out_specs=[nodes2, nodes1, nodes1],
        out_shape=_DENSE_IN_OUT,
    )(num1, den1, h2_sp, as_sp, ad_sp, b_sp,
      num2, den2, h2_gs, as_gs, ad_gs, b_gs,
      W_f, a_src_f, a_dst_f)


def _dense_out_body(num_ref, den_ref, h2_ref, as_ref, ad_ref, b_ref, o_ref):
    o_ref[...] = _epilogue(num_ref, den_ref, h2_ref, as_ref, ad_ref, b_ref,
                           False)


def _dense_out(num, den, h2, a_s, a_d, bias):
    return pl.pallas_call(
        _dense_out_body, out_shape=jax.ShapeDtypeStruct((N, D), jnp.float32),
    )(num, den, h2, a_s, a_d, bias)


def _sc_agg_body(h2f_hbm, as_hbm, ad_hbm, src_hbm, dst_hbm,
                 numf_hbm, den_hbm,
                 src_all, dst_all, as_t, ad_t,
                 rows_g0, rows_g1, rows_g2, rows_s0, rows_s1, rows_s2,
                 idxg0, idxg1, idxg2, idxs0, idxs1, idxs2,
                 wb0, wb1, wb2, wsc0, wsc1, wsc2,
                 num_sp, den_sp,
                 gsem0, gsem1, gsem2, ssem0, ssem1, ssem2,
                 dsem0, dsem1, dsem2):
    cid = lax.axis_index("c")
    sid = lax.axis_index("s")

    rows_g = (rows_g0, rows_g1, rows_g2)
    rows_s = (rows_s0, rows_s1, rows_s2)
    idxg = (idxg0, idxg1, idxg2)
    idxs = (idxs0, idxs1, idxs2)
    wb = (wb0, wb1, wb2)
    wsc = (wsc0, wsc1, wsc2)
    gsem = (gsem0, gsem1, gsem2)
    ssem = (ssem0, ssem1, ssem2)
    dsem = (dsem0, dsem1, dsem2)

    zero16 = jnp.zeros((16,), jnp.float32)
    lane = lax.iota(jnp.int32, 16)
    nvec = B // 16
    ebase = sid * EPT_PAD

    pltpu.sync_copy(src_hbm.at[pl.ds(ebase, EPT_PAD)], src_all)
    pltpu.sync_copy(dst_hbm.at[pl.ds(ebase, EPT_PAD)], dst_all)
    pltpu.sync_copy(as_hbm, as_t)
    pltpu.sync_copy(ad_hbm, ad_t)

    @pl.loop(0, B)
    def _zrow(r):
        for f in range(4):
            rows_g0[r, pl.ds(16 * f, 16)] = zero16

    for j in range(nvec):
        wb0[pl.ds(16 * j, 16)] = zero16

    row0 = sid * ROWS_PER_TILE
    for k in range(ROWS_PER_TILE // B):
        pltpu.sync_copy(rows_g0, num_sp.at[pl.ds(row0 + k * B, B)])
        pltpu.sync_copy(wb0, den_sp.at[pl.ds(row0 + k * B, B)])
    plsc.subcore_barrier()

    coff = jnp.full((16,), cid * N, jnp.int32)

    def prep(b, c):
        for j in range(nvec):
            sl = pl.ds(c * B + 16 * j, 16)
            sv = src_all[sl]
            dv = dst_all[sl]
            idxg[b][pl.ds(16 * j, 16)] = sv + coff
            e = plsc.load_gather(as_t, [sv]) + plsc.load_gather(ad_t, [dv])
            e = jnp.where(e >= 0.0, e, e * NEG_SLOPE)
            w = jnp.exp(e)
            valid = (lane + (c * B + 16 * j)) < EDGES_PER_TILE
            wb[b][pl.ds(16 * j, 16)] = jnp.where(valid, w, 0.0)
        pltpu.async_copy(h2f_hbm.at[idxg[b]], rows_g[b], gsem[b])

    def wait_gather(b):
        pltpu.make_async_copy(h2f_hbm.at[idxg[b]], rows_g[b], gsem[b]).wait()

    def wait_scatter(b):
        pltpu.make_async_copy(rows_s[b], num_sp.at[idxs[b]], ssem[b]).wait()

        @pl.when(cid == 0)
        def _():
            pltpu.make_async_copy(wsc[b], den_sp.at[idxs[b]], dsem[b]).wait()

    def process(b, c):
        for j in range(nvec):
            idxs[b][pl.ds(16 * j, 16)] = dst_all[pl.ds(c * B + 16 * j, 16)]
        for j in range(nvec):
            wv = wb[b][pl.ds(16 * j, 16)]
            for r in range(16):
                wr = jnp.full((16,), wv[r])
                row = 16 * j + r
                for f in range(4):
                    sl = pl.ds(16 * f, 16)
                    rows_s[b][row, sl] = rows_g[b][row, sl] * wr
        pltpu.async_copy(rows_s[b], num_sp.at[idxs[b]], ssem[b], add=True)

        @pl.when(cid == 0)
        def _():
            for j in range(nvec):
                sl = pl.ds(16 * j, 16)
                wsc[b][sl] = wb[b][sl]
            pltpu.async_copy(wsc[b], den_sp.at[idxs[b]], dsem[b], add=True)

    for b in range(NBUF):
        prep(b, b)

    for b in range(NBUF):
        wait_gather(b)
        process(b, b)
        prep(b, b + NBUF)

    @pl.loop(1, (NCHUNK - NBUF - 1) // NBUF)
    def _grp(k):
        for b in range(NBUF):
            c = NBUF * k + b
            wait_gather(b)
            wait_scatter(b)
            process(b, c)
            prep(b, c + NBUF)

    ktail = (NCHUNK - NBUF - 1) // NBUF
    for c in range(ktail * NBUF, NCHUNK):
        b = c % NBUF
        wait_gather(b)
        wait_scatter(b)
        process(b, c)
        if c + NBUF < NCHUNK:
            prep(b, c + NBUF)

    for b in range(NBUF):
        wait_scatter(b)

    plsc.subcore_barrier()

    out0 = cid * N + row0

    @pl.when(sid < NSUB - 1)
    def _wfull():
        pltpu.sync_copy(num_sp.at[pl.ds(row0, ROWS_PER_TILE)],
                        numf_hbm.at[pl.ds(out0, ROWS_PER_TILE)])

    @pl.when(sid == NSUB - 1)
    def _wlast():
        pltpu.sync_copy(num_sp.at[pl.ds(row0, ROWS_LAST)],
                        numf_hbm.at[pl.ds(out0, ROWS_LAST)])

    @pl.when(cid == 0)
    def _wden():
        @pl.when(sid < NSUB - 1)
        def _dfull():
            pltpu.sync_copy(den_sp.at[pl.ds(row0, ROWS_PER_TILE)],
                            den_hbm.at[pl.ds(row0, ROWS_PER_TILE)])

        @pl.when(sid == NSUB - 1)
        def _dlast():
            pltpu.sync_copy(den_sp.at[pl.ds(row0, ROWS_LAST)],
                            den_hbm.at[pl.ds(row0, ROWS_LAST)])


def _sc_agg(h2, a_s, a_d, src, dst):
    mesh = plsc.VectorSubcoreMesh(core_axis_name="c", subcore_axis_name="s")
    h2f = h2.reshape(NCORE * N, DH)
    cp = pltpu.CompilerParams()
    if "needs_layout_passes" in pltpu.CompilerParams.__dataclass_fields__:
        cp = dataclasses.replace(cp, needs_layout_passes=False)
    if "use_tc_tiling_on_sc" in pltpu.CompilerParams.__dataclass_fields__:
        cp = dataclasses.replace(cp, use_tc_tiling_on_sc=False)
    kern = pl.kernel(
        _sc_agg_body,
        out_type=[
            jax.ShapeDtypeStruct((NCORE * N, DH), jnp.float32),
            jax.ShapeDtypeStruct((N,), jnp.float32),
        ],
        mesh=mesh,
        scratch_types=(
            [
                pltpu.VMEM((EPT_PAD,), jnp.int32),
                pltpu.VMEM((EPT_PAD,), jnp.int32),
                pltpu.VMEM((N,), jnp.float32),
                pltpu.VMEM((N,), jnp.float32),
            ]
            + [pltpu.VMEM((B, DH), jnp.float32)] * (2 * NBUF)
            + [pltpu.VMEM((B,), jnp.int32)] * (2 * NBUF)
            + [pltpu.VMEM((B,), jnp.float32)] * (2 * NBUF)
            + [
                pltpu.VMEM_SHARED((NPAD, DH), jnp.float32),
                pltpu.VMEM_SHARED((NPAD,), jnp.float32),
            ]
            + [pltpu.SemaphoreType.DMA] * (3 * NBUF)
        ),
        compiler_params=cp,
    )
    return kern(h2f, a_s, a_d, src, dst)


def _pad_edges(a):
    a2 = a.reshape(NSUB, EDGES_PER_TILE)
    a2 = jnp.pad(a2, ((0, 0), (0, EPT_PAD - EDGES_PER_TILE)))
    return a2.reshape(NSUB * EPT_PAD)


def _agg(h2, a_s, a_d, src, dst):
    num, den = _sc_agg(h2, a_s.reshape(N), a_d.reshape(N), src, dst)
    return num.reshape(NCORE, N, DH), den.reshape(N, 1)


def kernel(x, spatial_edge_index, gene_sim_edge_index,
           W_sp, a_src_sp, a_dst_sp, b_sp,
           W_gs, a_src_gs, a_dst_gs, b_gs,
           W_f, a_src_f, a_dst_f, b_f):
    sp_src = _pad_edges(spatial_edge_index[0])
    sp_dst = _pad_edges(spatial_edge_index[1])
    gs_src = _pad_edges(gene_sim_edge_index[0])
    gs_dst = _pad_edges(gene_sim_edge_index[1])

    h2_sp, as_sp, ad_sp, h2_gs, as_gs, ad_gs = _dense_in2(
        x, W_sp, a_src_sp, a_dst_sp, W_gs, a_src_gs, a_dst_gs)

    num1, den1 = _agg(h2_sp, as_sp, ad_sp, sp_src, sp_dst)
    num2, den2 = _agg(h2_gs, as_gs, ad_gs, gs_src, gs_dst)

    h2_f, as_f, ad_f = _dense_mid(num1, den1, h2_sp, as_sp, ad_sp, b_sp,
                                  num2, den2, h2_gs, as_gs, ad_gs, b_gs,
                                  W_f, a_src_f, a_dst_f)

    num3, den3 = _agg(h2_f, as_f, ad_f, sp_src, sp_dst)
    return _dense_out(num3, den3, h2_f, as_f, ad_f, b_f)


# speedup vs baseline: 1.3192x; 1.0273x over previous
import dataclasses
import functools

import jax
import jax.numpy as jnp
from jax import lax
from jax.experimental import pallas as pl
from jax.experimental.pallas import tpu as pltpu
from jax.experimental.pallas import tpu_sc as plsc

N = 10000
D = 128
DH = 64
E = 320000
NEG_SLOPE = 0.2

NSUB = 16
NCORE = 2
B = 64
EDGES_PER_TILE = E // NSUB
NCHUNK = -(-EDGES_PER_TILE // B)
EPT_PAD = NCHUNK * B
NBUF = 3
NPAD = 10240
ROWS_PER_TILE = NPAD // NSUB
ROWS_LAST = N - 15 * ROWS_PER_TILE


def _project(x, W_ref, asrc_ref, adst_ref, hlo_ref, hhi_ref, as_ref, ad_ref):
    h = jnp.dot(x, W_ref[...], preferred_element_type=jnp.float32)
    hlo_ref[...] = h[:, :DH]
    hhi_ref[...] = h[:, DH:]
    as_ref[...] = jnp.sum(h * asrc_ref[...][None, :], axis=1, keepdims=True)
    ad_ref[...] = jnp.sum(h * adst_ref[...][None, :], axis=1, keepdims=True)


def _epilogue(nlo_ref, nhi_ref, den_ref, hlo_ref, hhi_ref, as_ref, ad_ref,
              b_ref, use_elu):
    a = as_ref[...] + ad_ref[...]
    a = jnp.where(a >= 0.0, a, a * NEG_SLOPE)
    ws = jnp.exp(a)
    inv = 1.0 / (den_ref[...] + ws + 1e-16)
    b = b_ref[...]
    halves = []
    for c, (n_ref, h_ref) in enumerate(((nlo_ref, hlo_ref), (nhi_ref, hhi_ref))):
        o = (n_ref[...] + ws * h_ref[...]) * inv + b[c * DH:(c + 1) * DH][None, :]
        if use_elu:
            o = jnp.where(o > 0.0, o, jnp.exp(o) - 1.0)
        halves.append(o)
    return jnp.concatenate(halves, axis=1)


_H_OUT = [
    jax.ShapeDtypeStruct((N, DH), jnp.float32),
    jax.ShapeDtypeStruct((N, DH), jnp.float32),
    jax.ShapeDtypeStruct((N, 1), jnp.float32),
    jax.ShapeDtypeStruct((N, 1), jnp.float32),
]


def _dense_in2_body(x_ref, Wsp_ref, assp_ref, adsp_ref, Wgs_ref, asgs_ref,
                    adgs_ref, hsplo, hsphi, assp_o, adsp_o,
                    hgslo, hgshi, asgs_o, adgs_o):
    xv = x_ref[...]
    _project(xv, Wsp_ref, assp_ref, adsp_ref, hsplo, hsphi, assp_o, adsp_o)
    _project(xv, Wgs_ref, asgs_ref, adgs_ref, hgslo, hgshi, asgs_o, adgs_o)


def _dense_in2(x, W_sp, a_src_sp, a_dst_sp, W_gs, a_src_gs, a_dst_gs):
    return pl.pallas_call(_dense_in2_body, out_shape=_H_OUT * 2)(
        x, W_sp, a_src_sp, a_dst_sp, W_gs, a_src_gs, a_dst_gs)


def _dense_mid_body(n1lo, n1hi, den1_ref, h1lo, h1hi, assp_ref, adsp_ref,
                    bsp_ref,
                    n2lo, n2hi, den2_ref, h2lo, h2hi, asgs_ref, adgs_ref,
                    bgs_ref,
                    Wf_ref, asf_ref, adf_ref,
                    hflo, hfhi, asf_o, adf_o):
    o1 = _epilogue(n1lo, n1hi, den1_ref, h1lo, h1hi, assp_ref, adsp_ref,
                   bsp_ref, True)
    o2 = _epilogue(n2lo, n2hi, den2_ref, h2lo, h2hi, asgs_ref, adgs_ref,
                   bgs_ref, True)
    xf = (o1 + o2) * 0.5
    _project(xf, Wf_ref, asf_ref, adf_ref, hflo, hfhi, asf_o, adf_o)


_BN = 2000


def _dense_mid(n1lo, n1hi, den1, h1lo, h1hi, as_sp, ad_sp, b_sp,
               n2lo, n2hi, den2, h2lo, h2hi, as_gs, ad_gs, b_gs,
               W_f, a_src_f, a_dst_f):
    rows = pl.BlockSpec((_BN, DH), lambda i: (i, 0))
    col = pl.BlockSpec((_BN, 1), lambda i: (i, 0))
    full_w = pl.BlockSpec((D, D), lambda i: (0, 0))
    full_v = pl.BlockSpec((D,), lambda i: (0,))
    return pl.pallas_call(
        _dense_mid_body,
        grid=(N // _BN,),
        in_specs=[rows, rows, col, rows, rows, col, col, full_v,
                  rows, rows, col, rows, rows, col, col, full_v,
                  full_w, full_v, full_v],
        out_specs=[rows, rows, col, col],
        out_shape=_H_OUT,
    )(n1lo, n1hi, den1, h1lo, h1hi, as_sp, ad_sp, b_sp,
      n2lo, n2hi, den2, h2lo, h2hi, as_gs, ad_gs, b_gs,
      W_f, a_src_f, a_dst_f)


def _dense_out_body(nlo, nhi, den_ref, hlo, hhi, as_ref, ad_ref, b_ref, o_ref):
    o_ref[...] = _epilogue(nlo, nhi, den_ref, hlo, hhi, as_ref, ad_ref, b_ref,
                           False)


def _dense_out(nlo, nhi, den, hlo, hhi, a_s, a_d, bias):
    return pl.pallas_call(
        _dense_out_body, out_shape=jax.ShapeDtypeStruct((N, D), jnp.float32),
    )(nlo, nhi, den, hlo, hhi, a_s, a_d, bias)


def _sc_agg_body(hlo_hbm, hhi_hbm, as_hbm, ad_hbm, edge_hbm,
                 numlo_hbm, numhi_hbm, den_hbm,
                 src_all, dst_all, as_t, ad_t,
                 rows_g0, rows_g1, rows_g2, rows_s0, rows_s1, rows_s2,
                 idxs0, idxs1, idxs2,
                 wb0, wb1, wb2, wsc0, wsc1, wsc2,
                 num_sp, den_sp,
                 gsem0, gsem1, gsem2, ssem0, ssem1, ssem2,
                 dsem0, dsem1, dsem2):
    cid = lax.axis_index("c")
    sid = lax.axis_index("s")

    rows_g = (rows_g0, rows_g1, rows_g2)
    rows_s = (rows_s0, rows_s1, rows_s2)
    idxs = (idxs0, idxs1, idxs2)
    wb = (wb0, wb1, wb2)
    wsc = (wsc0, wsc1, wsc2)
    gsem = (gsem0, gsem1, gsem2)
    ssem = (ssem0, ssem1, ssem2)
    dsem = (dsem0, dsem1, dsem2)

    zero16 = jnp.zeros((16,), jnp.float32)
    zero16i = jnp.zeros((16,), jnp.int32)
    lane = lax.iota(jnp.int32, 16)
    nvec = B // 16
    ebase = sid * EDGES_PER_TILE

    pltpu.sync_copy(edge_hbm.at[0, pl.ds(ebase, EDGES_PER_TILE)],
                    src_all.at[pl.ds(0, EDGES_PER_TILE)])
    pltpu.sync_copy(edge_hbm.at[1, pl.ds(ebase, EDGES_PER_TILE)],
                    dst_all.at[pl.ds(0, EDGES_PER_TILE)])
    pltpu.sync_copy(as_hbm, as_t)
    pltpu.sync_copy(ad_hbm, ad_t)
    for t in range((EPT_PAD - EDGES_PER_TILE) // 16):
        src_all[pl.ds(EDGES_PER_TILE + 16 * t, 16)] = zero16i
        dst_all[pl.ds(EDGES_PER_TILE + 16 * t, 16)] = zero16i

    @pl.loop(0, B)
    def _zrow(r):
        for f in range(4):
            rows_g0[r, pl.ds(16 * f, 16)] = zero16

    for j in range(nvec):
        wb0[pl.ds(16 * j, 16)] = zero16

    row0 = sid * ROWS_PER_TILE
    for k in range(ROWS_PER_TILE // B):
        pltpu.sync_copy(rows_g0, num_sp.at[pl.ds(row0 + k * B, B)])
        pltpu.sync_copy(wb0, den_sp.at[pl.ds(row0 + k * B, B)])
    plsc.subcore_barrier()

    def prep(b, c):
        for j in range(nvec):
            sl = pl.ds(c * B + 16 * j, 16)
            sv = src_all[sl]
            dv = dst_all[sl]
            e = plsc.load_gather(as_t, [sv]) + plsc.load_gather(ad_t, [dv])
            e = jnp.where(e >= 0.0, e, e * NEG_SLOPE)
            w = jnp.exp(e)
            valid = (lane + (c * B + 16 * j)) < EDGES_PER_TILE
            wb[b][pl.ds(16 * j, 16)] = jnp.where(valid, w, 0.0)
        gidx = src_all.at[pl.ds(c * B, B)]

        @pl.when(cid == 0)
        def _():
            pltpu.async_copy(hlo_hbm.at[gidx], rows_g[b], gsem[b])

        @pl.when(cid == 1)
        def _():
            pltpu.async_copy(hhi_hbm.at[gidx], rows_g[b], gsem[b])

    def wait_gather(b, c):
        gidx = src_all.at[pl.ds(c * B, B)]

        @pl.when(cid == 0)
        def _():
            pltpu.make_async_copy(hlo_hbm.at[gidx], rows_g[b], gsem[b]).wait()

        @pl.when(cid == 1)
        def _():
            pltpu.make_async_copy(hhi_hbm.at[gidx], rows_g[b], gsem[b]).wait()

    def wait_scatter(b):
        pltpu.make_async_copy(rows_s[b], num_sp.at[idxs[b]], ssem[b]).wait()

        @pl.when(cid == 0)
        def _():
            pltpu.make_async_copy(wsc[b], den_sp.at[idxs[b]], dsem[b]).wait()

    def process(b, c):
        for j in range(nvec):
            idxs[b][pl.ds(16 * j, 16)] = dst_all[pl.ds(c * B + 16 * j, 16)]
        for j in range(nvec):
            wv = wb[b][pl.ds(16 * j, 16)]
            for r in range(16):
                wr = jnp.full((16,), wv[r])
                row = 16 * j + r
                for f in range(4):
                    sl = pl.ds(16 * f, 16)
                    rows_s[b][row, sl] = rows_g[b][row, sl] * wr
        pltpu.async_copy(rows_s[b], num_sp.at[idxs[b]], ssem[b], add=True)

        @pl.when(cid == 0)
        def _():
            for j in range(nvec):
                sl = pl.ds(16 * j, 16)
                wsc[b][sl] = wb[b][sl]
            pltpu.async_copy(wsc[b], den_sp.at[idxs[b]], dsem[b], add=True)

    for b in range(NBUF):
        prep(b, b)

    for b in range(NBUF):
        wait_gather(b, b)
        process(b, b)
        prep(b, b + NBUF)

    @pl.loop(1, (NCHUNK - NBUF - 1) // NBUF)
    def _grp(k):
        for b in range(NBUF):
            c = NBUF * k + b
            wait_gather(b, c)
            wait_scatter(b)
            process(b, c)
            prep(b, c + NBUF)

    ktail = (NCHUNK - NBUF - 1) // NBUF
    for c in range(ktail * NBUF, NCHUNK):
        b = c % NBUF
        wait_gather(b, c)
        wait_scatter(b)
        process(b, c)
        if c + NBUF < NCHUNK:
            prep(b, c + NBUF)

    for b in range(NBUF):
        wait_scatter(b)

    plsc.subcore_barrier()

    def _writeout(dst_hbm, src_sp):
        @pl.when(sid < NSUB - 1)
        def _full():
            pltpu.sync_copy(src_sp.at[pl.ds(row0, ROWS_PER_TILE)],
                            dst_hbm.at[pl.ds(row0, ROWS_PER_TILE)])

        @pl.when(sid == NSUB - 1)
        def _last():
            pltpu.sync_copy(src_sp.at[pl.ds(row0, ROWS_LAST)],
                            dst_hbm.at[pl.ds(row0, ROWS_LAST)])

    @pl.when(cid == 0)
    def _wlo():
        _writeout(numlo_hbm, num_sp)
        _writeout(den_hbm, den_sp)

    @pl.when(cid == 1)
    def _whi():
        _writeout(numhi_hbm, num_sp)


def _sc_agg(hlo, hhi, a_s, a_d, edge_index):
    mesh = plsc.VectorSubcoreMesh(core_axis_name="c", subcore_axis_name="s")
    cp = pltpu.CompilerParams()
    if "needs_layout_passes" in pltpu.CompilerParams.__dataclass_fields__:
        cp = dataclasses.replace(cp, needs_layout_passes=False)
    if "use_tc_tiling_on_sc" in pltpu.CompilerParams.__dataclass_fields__:
        cp = dataclasses.replace(cp, use_tc_tiling_on_sc=False)
    kern = pl.kernel(
        _sc_agg_body,
        out_type=[
            jax.ShapeDtypeStruct((N, DH), jnp.float32),
            jax.ShapeDtypeStruct((N, DH), jnp.float32),
            jax.ShapeDtypeStruct((N,), jnp.float32),
        ],
        mesh=mesh,
        scratch_types=(
            [
                pltpu.VMEM((EPT_PAD,), jnp.int32),
                pltpu.VMEM((EPT_PAD,), jnp.int32),
                pltpu.VMEM((N,), jnp.float32),
                pltpu.VMEM((N,), jnp.float32),
            ]
            + [pltpu.VMEM((B, DH), jnp.float32)] * (2 * NBUF)
            + [pltpu.VMEM((B,), jnp.int32)] * NBUF
            + [pltpu.VMEM((B,), jnp.float32)] * (2 * NBUF)
            + [
                pltpu.VMEM_SHARED((NPAD, DH), jnp.float32),
                pltpu.VMEM_SHARED((NPAD,), jnp.float32),
            ]
            + [pltpu.SemaphoreType.DMA] * (3 * NBUF)
        ),
        compiler_params=cp,
    )
    return kern(hlo, hhi, a_s, a_d, edge_index)


def _agg(hlo, hhi, a_s, a_d, edge_index):
    nlo, nhi, den = _sc_agg(hlo, hhi, a_s.reshape(N), a_d.reshape(N),
                            edge_index)
    return nlo, nhi, den.reshape(N, 1)


def kernel(x, spatial_edge_index, gene_sim_edge_index,
           W_sp, a_src_sp, a_dst_sp, b_sp,
           W_gs, a_src_gs, a_dst_gs, b_gs,
           W_f, a_src_f, a_dst_f, b_f):
    h1lo, h1hi, as_sp, ad_sp, h2lo, h2hi, as_gs, ad_gs = _dense_in2(
        x, W_sp, a_src_sp, a_dst_sp, W_gs, a_src_gs, a_dst_gs)

    n1lo, n1hi, den1 = _agg(h1lo, h1hi, as_sp, ad_sp, spatial_edge_index)
    n2lo, n2hi, den2 = _agg(h2lo, h2hi, as_gs, ad_gs, gene_sim_edge_index)

    hflo, hfhi, as_f, ad_f = _dense_mid(
        n1lo, n1hi, den1, h1lo, h1hi, as_sp, ad_sp, b_sp,
        n2lo, n2hi, den2, h2lo, h2hi, as_gs, ad_gs, b_gs,
        W_f, a_src_f, a_dst_f)

    n3lo, n3hi, den3 = _agg(hflo, hfhi, as_f, ad_f, spatial_edge_index)
    return _dense_out(n3lo, n3hi, den3, hflo, hfhi, as_f, ad_f, b_f)


# speedup vs baseline: 1.3549x; 1.0271x over previous
import dataclasses
import functools

import jax
import jax.numpy as jnp
from jax import lax
from jax.experimental import pallas as pl
from jax.experimental.pallas import tpu as pltpu
from jax.experimental.pallas import tpu_sc as plsc

N = 10000
D = 128
DH = 64
E = 320000
NEG_SLOPE = 0.2

NSUB = 16
NCORE = 2
B = 64
EDGES_PER_TILE = E // NSUB
NCHUNK = -(-EDGES_PER_TILE // B)
EPT_PAD = NCHUNK * B
NBUF = 3
NPAD = 10240
ROWS_PER_TILE = NPAD // NSUB
ROWS_LAST = N - 15 * ROWS_PER_TILE


def _project(x, W_ref, asrc_ref, adst_ref, hlo_ref, hhi_ref, as_ref, ad_ref):
    h = jnp.dot(x, W_ref[...], preferred_element_type=jnp.float32)
    hlo_ref[...] = h[:, :DH]
    hhi_ref[...] = h[:, DH:]
    as_ref[...] = jnp.sum(h * asrc_ref[...][None, :], axis=1, keepdims=True)
    ad_ref[...] = jnp.sum(h * adst_ref[...][None, :], axis=1, keepdims=True)


def _epilogue(nlo_ref, nhi_ref, den_ref, hlo_ref, hhi_ref, as_ref, ad_ref,
              b_ref, use_elu):
    a = as_ref[...] + ad_ref[...]
    a = jnp.where(a >= 0.0, a, a * NEG_SLOPE)
    ws = jnp.exp(a)
    inv = 1.0 / (den_ref[...] + ws + 1e-16)
    b = b_ref[...]
    halves = []
    for c, (n_ref, h_ref) in enumerate(((nlo_ref, hlo_ref), (nhi_ref, hhi_ref))):
        o = (n_ref[...] + ws * h_ref[...]) * inv + b[c * DH:(c + 1) * DH][None, :]
        if use_elu:
            o = jnp.where(o > 0.0, o, jnp.exp(o) - 1.0)
        halves.append(o)
    return jnp.concatenate(halves, axis=1)


_H_OUT = [
    jax.ShapeDtypeStruct((N, DH), jnp.float32),
    jax.ShapeDtypeStruct((N, DH), jnp.float32),
    jax.ShapeDtypeStruct((N, 1), jnp.float32),
    jax.ShapeDtypeStruct((N, 1), jnp.float32),
]


def _dense_in2_body(x_ref, Wsp_ref, assp_ref, adsp_ref, Wgs_ref, asgs_ref,
                    adgs_ref, hsplo, hsphi, assp_o, adsp_o,
                    hgslo, hgshi, asgs_o, adgs_o):
    xv = x_ref[...]
    _project(xv, Wsp_ref, assp_ref, adsp_ref, hsplo, hsphi, assp_o, adsp_o)
    _project(xv, Wgs_ref, asgs_ref, adgs_ref, hgslo, hgshi, asgs_o, adgs_o)


def _dense_in2(x, W_sp, a_src_sp, a_dst_sp, W_gs, a_src_gs, a_dst_gs):
    return pl.pallas_call(_dense_in2_body, out_shape=_H_OUT * 2)(
        x, W_sp, a_src_sp, a_dst_sp, W_gs, a_src_gs, a_dst_gs)


def _dense_mid_body(n1lo, n1hi, den1_ref, h1lo, h1hi, assp_ref, adsp_ref,
                    bsp_ref,
                    n2lo, n2hi, den2_ref, h2lo, h2hi, asgs_ref, adgs_ref,
                    bgs_ref,
                    Wf_ref, asf_ref, adf_ref,
                    hflo, hfhi, asf_o, adf_o):
    o1 = _epilogue(n1lo, n1hi, den1_ref, h1lo, h1hi, assp_ref, adsp_ref,
                   bsp_ref, True)
    o2 = _epilogue(n2lo, n2hi, den2_ref, h2lo, h2hi, asgs_ref, adgs_ref,
                   bgs_ref, True)
    xf = (o1 + o2) * 0.5
    _project(xf, Wf_ref, asf_ref, adf_ref, hflo, hfhi, asf_o, adf_o)


_BN = 2000


def _dense_mid(n1lo, n1hi, den1, h1lo, h1hi, as_sp, ad_sp, b_sp,
               n2lo, n2hi, den2, h2lo, h2hi, as_gs, ad_gs, b_gs,
               W_f, a_src_f, a_dst_f):
    rows = pl.BlockSpec((_BN, DH), lambda i: (i, 0))
    col = pl.BlockSpec((_BN, 1), lambda i: (i, 0))
    full_w = pl.BlockSpec((D, D), lambda i: (0, 0))
    full_v = pl.BlockSpec((D,), lambda i: (0,))
    return pl.pallas_call(
        _dense_mid_body,
        grid=(N // _BN,),
        in_specs=[rows, rows, col, rows, rows, col, col, full_v,
                  rows, rows, col, rows, rows, col, col, full_v,
                  full_w, full_v, full_v],
        out_specs=[rows, rows, col, col],
        out_shape=_H_OUT,
    )(n1lo, n1hi, den1, h1lo, h1hi, as_sp, ad_sp, b_sp,
      n2lo, n2hi, den2, h2lo, h2hi, as_gs, ad_gs, b_gs,
      W_f, a_src_f, a_dst_f)


def _dense_out_body(nlo, nhi, den_ref, hlo, hhi, as_ref, ad_ref, b_ref, o_ref):
    o_ref[...] = _epilogue(nlo, nhi, den_ref, hlo, hhi, as_ref, ad_ref, b_ref,
                           False)


def _dense_out(nlo, nhi, den, hlo, hhi, a_s, a_d, bias):
    return pl.pallas_call(
        _dense_out_body, out_shape=jax.ShapeDtypeStruct((N, D), jnp.float32),
    )(nlo, nhi, den, hlo, hhi, a_s, a_d, bias)


def _sc_agg_body(hlo_hbm, hhi_hbm, as_hbm, ad_hbm, edge_hbm,
                 numlo_hbm, numhi_hbm, den_hbm,
                 src_all, dst_all, as_t, ad_t,
                 rows_g0, rows_g1, rows_g2, rows_s0, rows_s1, rows_s2,
                 idxs0, idxs1, idxs2,
                 wb0, wb1, wb2, wsc0, wsc1, wsc2,
                 num_sp, den_sp,
                 gsem0, gsem1, gsem2, ssem0, ssem1, ssem2,
                 dsem0, dsem1, dsem2):
    cid = lax.axis_index("c")
    sid = lax.axis_index("s")

    rows_g = (rows_g0, rows_g1, rows_g2)
    rows_s = (rows_s0, rows_s1, rows_s2)
    idxs = (idxs0, idxs1, idxs2)
    wb = (wb0, wb1, wb2)
    wsc = (wsc0, wsc1, wsc2)
    gsem = (gsem0, gsem1, gsem2)
    ssem = (ssem0, ssem1, ssem2)
    dsem = (dsem0, dsem1, dsem2)

    zero16 = jnp.zeros((16,), jnp.float32)
    zero16i = jnp.zeros((16,), jnp.int32)
    lane = lax.iota(jnp.int32, 16)
    nvec = B // 16
    ebase = sid * EDGES_PER_TILE

    pltpu.sync_copy(edge_hbm.at[0, pl.ds(ebase, EDGES_PER_TILE)],
                    src_all.at[pl.ds(0, EDGES_PER_TILE)])
    pltpu.sync_copy(edge_hbm.at[1, pl.ds(ebase, EDGES_PER_TILE)],
                    dst_all.at[pl.ds(0, EDGES_PER_TILE)])
    pltpu.sync_copy(as_hbm, as_t)
    pltpu.sync_copy(ad_hbm, ad_t)
    for t in range((EPT_PAD - EDGES_PER_TILE) // 16):
        src_all[pl.ds(EDGES_PER_TILE + 16 * t, 16)] = zero16i
        dst_all[pl.ds(EDGES_PER_TILE + 16 * t, 16)] = zero16i

    @pl.loop(0, B)
    def _zrow(r):
        for f in range(4):
            rows_g0[r, pl.ds(16 * f, 16)] = zero16

    for j in range(nvec):
        wb0[pl.ds(16 * j, 16)] = zero16

    row0 = sid * ROWS_PER_TILE
    for k in range(ROWS_PER_TILE // B):
        pltpu.sync_copy(rows_g0, num_sp.at[pl.ds(row0 + k * B, B)])
        pltpu.sync_copy(wb0, den_sp.at[pl.ds(row0 + k * B, B)])
    plsc.subcore_barrier()

    def prep(b, c):
        gidx = src_all.at[pl.ds(c * B, B)]

        @pl.when(cid == 0)
        def _():
            pltpu.async_copy(hlo_hbm.at[gidx], rows_g[b], gsem[b])

        @pl.when(cid == 1)
        def _():
            pltpu.async_copy(hhi_hbm.at[gidx], rows_g[b], gsem[b])

        for j in range(nvec):
            sl = pl.ds(c * B + 16 * j, 16)
            sv = src_all[sl]
            dv = dst_all[sl]
            e = plsc.load_gather(as_t, [sv]) + plsc.load_gather(ad_t, [dv])
            e = jnp.where(e >= 0.0, e, e * NEG_SLOPE)
            w = jnp.exp(e)
            valid = (lane + (c * B + 16 * j)) < EDGES_PER_TILE
            wb[b][pl.ds(16 * j, 16)] = jnp.where(valid, w, 0.0)

    def wait_gather(b, c):
        gidx = src_all.at[pl.ds(c * B, B)]

        @pl.when(cid == 0)
        def _():
            pltpu.make_async_copy(hlo_hbm.at[gidx], rows_g[b], gsem[b]).wait()

        @pl.when(cid == 1)
        def _():
            pltpu.make_async_copy(hhi_hbm.at[gidx], rows_g[b], gsem[b]).wait()

    def wait_scatter(b):
        pltpu.make_async_copy(rows_s[b], num_sp.at[idxs[b]], ssem[b]).wait()

        @pl.when(cid == 0)
        def _():
            pltpu.make_async_copy(wsc[b], den_sp.at[idxs[b]], dsem[b]).wait()

    def process(b, c):
        for j in range(nvec):
            idxs[b][pl.ds(16 * j, 16)] = dst_all[pl.ds(c * B + 16 * j, 16)]
        for j in range(nvec):
            wv = wb[b][pl.ds(16 * j, 16)]
            for r in range(16):
                wr = jnp.full((16,), wv[r])
                row = 16 * j + r
                for f in range(4):
                    sl = pl.ds(16 * f, 16)
                    rows_s[b][row, sl] = rows_g[b][row, sl] * wr
        pltpu.async_copy(rows_s[b], num_sp.at[idxs[b]], ssem[b], add=True)

        @pl.when(cid == 0)
        def _():
            for j in range(nvec):
                sl = pl.ds(16 * j, 16)
                wsc[b][sl] = wb[b][sl]
            pltpu.async_copy(wsc[b], den_sp.at[idxs[b]], dsem[b], add=True)

    for b in range(NBUF):
        prep(b, b)

    for b in range(NBUF):
        wait_gather(b, b)
        process(b, b)
        prep(b, b + NBUF)

    @pl.loop(1, (NCHUNK - NBUF - 1) // NBUF)
    def _grp(k):
        for b in range(NBUF):
            c = NBUF * k + b
            wait_gather(b, c)
            wait_scatter(b)
            process(b, c)
            prep(b, c + NBUF)

    ktail = (NCHUNK - NBUF - 1) // NBUF
    for c in range(ktail * NBUF, NCHUNK):
        b = c % NBUF
        wait_gather(b, c)
        wait_scatter(b)
        process(b, c)
        if c + NBUF < NCHUNK:
            prep(b, c + NBUF)

    for b in range(NBUF):
        wait_scatter(b)

    plsc.subcore_barrier()

    def _writeout(dst_hbm, src_sp):
        @pl.when(sid < NSUB - 1)
        def _full():
            pltpu.sync_copy(src_sp.at[pl.ds(row0, ROWS_PER_TILE)],
                            dst_hbm.at[pl.ds(row0, ROWS_PER_TILE)])

        @pl.when(sid == NSUB - 1)
        def _last():
            pltpu.sync_copy(src_sp.at[pl.ds(row0, ROWS_LAST)],
                            dst_hbm.at[pl.ds(row0, ROWS_LAST)])

    @pl.when(cid == 0)
    def _wlo():
        _writeout(numlo_hbm, num_sp)
        _writeout(den_hbm, den_sp)

    @pl.when(cid == 1)
    def _whi():
        _writeout(numhi_hbm, num_sp)


def _sc_agg(hlo, hhi, a_s, a_d, edge_index):
    mesh = plsc.VectorSubcoreMesh(core_axis_name="c", subcore_axis_name="s")
    cp = pltpu.CompilerParams()
    if "needs_layout_passes" in pltpu.CompilerParams.__dataclass_fields__:
        cp = dataclasses.replace(cp, needs_layout_passes=False)
    if "use_tc_tiling_on_sc" in pltpu.CompilerParams.__dataclass_fields__:
        cp = dataclasses.replace(cp, use_tc_tiling_on_sc=False)
    kern = pl.kernel(
        _sc_agg_body,
        out_type=[
            jax.ShapeDtypeStruct((N, DH), jnp.float32),
            jax.ShapeDtypeStruct((N, DH), jnp.float32),
            jax.ShapeDtypeStruct((N,), jnp.float32),
        ],
        mesh=mesh,
        scratch_types=(
            [
                pltpu.VMEM((EPT_PAD,), jnp.int32),
                pltpu.VMEM((EPT_PAD,), jnp.int32),
                pltpu.VMEM((N,), jnp.float32),
                pltpu.VMEM((N,), jnp.float32),
            ]
            + [pltpu.VMEM((B, DH), jnp.float32)] * (2 * NBUF)
            + [pltpu.VMEM((B,), jnp.int32)] * NBUF
            + [pltpu.VMEM((B,), jnp.float32)] * (2 * NBUF)
            + [
                pltpu.VMEM_SHARED((NPAD, DH), jnp.float32),
                pltpu.VMEM_SHARED((NPAD,), jnp.float32),
            ]
            + [pltpu.SemaphoreType.DMA] * (3 * NBUF)
        ),
        compiler_params=cp,
    )
    return kern(hlo, hhi, a_s, a_d, edge_index)


def _agg(hlo, hhi, a_s, a_d, edge_index):
    nlo, nhi, den = _sc_agg(hlo, hhi, a_s.reshape(N), a_d.reshape(N),
                            edge_index)
    return nlo, nhi, den.reshape(N, 1)


def kernel(x, spatial_edge_index, gene_sim_edge_index,
           W_sp, a_src_sp, a_dst_sp, b_sp,
           W_gs, a_src_gs, a_dst_gs, b_gs,
           W_f, a_src_f, a_dst_f, b_f):
    h1lo, h1hi, as_sp, ad_sp, h2lo, h2hi, as_gs, ad_gs = _dense_in2(
        x, W_sp, a_src_sp, a_dst_sp, W_gs, a_src_gs, a_dst_gs)

    n1lo, n1hi, den1 = _agg(h1lo, h1hi, as_sp, ad_sp, spatial_edge_index)
    n2lo, n2hi, den2 = _agg(h2lo, h2hi, as_gs, ad_gs, gene_sim_edge_index)

    hflo, hfhi, as_f, ad_f = _dense_mid(
        n1lo, n1hi, den1, h1lo, h1hi, as_sp, ad_sp, b_sp,
        n2lo, n2hi, den2, h2lo, h2hi, as_gs, ad_gs, b_gs,
        W_f, a_src_f, a_dst_f)

    n3lo, n3hi, den3 = _agg(hflo, hfhi, as_f, ad_f, spatial_edge_index)
    return _dense_out(n3lo, n3hi, den3, hflo, hfhi, as_f, ad_f, b_f)
